# baseline (speedup 1.0000x reference)
import jax
import jax.numpy as jnp
from jax import lax
from jax.experimental import pallas as pl
from jax.experimental.pallas import tpu as pltpu
from jax.experimental.pallas import tpu_sc as plsc

N = 10000
NPAD = 10240
E = 320000
D = 128
NC, NS = 2, 16
TILES = NC * NS
BLK = 80
BPT = 128
EPAD = TILES * BPT * BLK
ROWS_PER_TILE = NPAD // NS


def _sc_mesh():
    return plsc.VectorSubcoreMesh(core_axis_name="c", subcore_axis_name="s")


def _deg_kernel_body(dst_hbm, out_hbm, hist_v, idx_v):
    c = lax.axis_index("c")
    s = lax.axis_index("s")
    wid = c * NS + s

    pltpu.sync_copy(dst_hbm.at[pl.ds(wid * BPT, BPT)], idx_v)

    @pl.loop(0, NPAD // 16)
    def _(i):
        hist_v[pl.ds(i * 16, 16)] = jnp.zeros((16,), jnp.float32)

    @pl.loop(0, BPT)
    def _(b):
        for j in range(BLK // 16):
            idx = idx_v[b, pl.ds(j * 16, 16)]
            cnt, last = plsc.scan_count(idx)
            plsc.addupdate_scatter(hist_v, [idx], cnt.astype(jnp.float32),
                                   mask=last)

    pltpu.sync_copy(hist_v, out_hbm.at[pl.ds(wid * NPAD, NPAD)])


def _deg(dst3):
    fn = pl.kernel(
        _deg_kernel_body,
        out_type=jax.ShapeDtypeStruct((TILES * NPAD,), jnp.float32),
        mesh=_sc_mesh(),
        compiler_params=pltpu.CompilerParams(needs_layout_passes=False),
        scratch_types=[
            pltpu.VMEM((NPAD,), jnp.float32),
            pltpu.VMEM((BPT, BLK), jnp.int32),
        ],
    )
    return fn(dst3)


def _agg_kernel_body(g_hbm, src_hbm, dst_hbm, zeros_hbm, out_hbm,
                     acc, sidx, didx, rows0, sem0, sems):
    c = lax.axis_index("c")
    s = lax.axis_index("s")
    wid = c * NS + s
    row0 = s * ROWS_PER_TILE

    pltpu.sync_copy(src_hbm.at[pl.ds(wid * BPT, BPT)], sidx)
    pltpu.sync_copy(dst_hbm.at[pl.ds(wid * BPT, BPT)], didx)

    @pl.when(c == 0)
    def _():
        @pl.loop(0, ROWS_PER_TILE // BLK)
        def _(k):
            pltpu.sync_copy(g_hbm.at[pl.ds(row0 + k * BLK, BLK)], rows0.at[pl.ds(0, BLK)])
            pltpu.sync_copy(rows0.at[pl.ds(0, BLK)], acc.at[pl.ds(row0 + k * BLK, BLK)])

    @pl.when(c == 1)
    def _():
        pltpu.sync_copy(zeros_hbm, rows0.at[pl.ds(0, BLK)])

        @pl.loop(0, ROWS_PER_TILE // BLK)
        def _(k):
            pltpu.sync_copy(rows0.at[pl.ds(0, BLK)], acc.at[pl.ds(row0 + k * BLK, BLK)])

    plsc.subcore_barrier()

    @pl.loop(0, BPT)
    def _(b):
        cur = lax.rem(b, 2) * BLK
        prv = (1 - lax.rem(b, 2)) * BLK
        pltpu.async_copy(g_hbm.at[sidx.at[b]], rows0.at[pl.ds(cur, BLK)],
                         sem0).wait()

        del prv

        @pl.when(b > 0)
        def _():
            pltpu.make_async_copy(zeros_hbm, acc.at[pl.ds(0, BLK)], sems).wait()

        pltpu.async_copy(rows0.at[pl.ds(cur, BLK)], acc.at[didx.at[b]],
                         sems, add=True)

    pltpu.make_async_copy(zeros_hbm, acc.at[pl.ds(0, BLK)], sems).wait()

    plsc.subcore_barrier()

    @pl.loop(0, ROWS_PER_TILE // BLK)
    def _(k):
        pltpu.sync_copy(acc.at[pl.ds(row0 + k * BLK, BLK)], rows0.at[pl.ds(0, BLK)])
        pltpu.sync_copy(rows0.at[pl.ds(0, BLK)], out_hbm.at[pl.ds(c * NPAD + row0 + k * BLK, BLK)])


def _agg(g, src3, dst3, zeros_tile):
    fn = pl.kernel(
        _agg_kernel_body,
        out_type=jax.ShapeDtypeStruct((NC * NPAD, D), jnp.float32),
        mesh=_sc_mesh(),
        compiler_params=pltpu.CompilerParams(use_tc_tiling_on_sc=False),
        scratch_types=[
            pltpu.VMEM_SHARED((NPAD, D), jnp.float32),
            pltpu.VMEM((BPT, BLK), jnp.int32),
            pltpu.VMEM((BPT, BLK), jnp.int32),
            pltpu.VMEM((2 * BLK, D), jnp.float32),
            pltpu.SemaphoreType.DMA,
            pltpu.SemaphoreType.DMA,
        ],
    )
    return fn(g, src3, dst3, zeros_tile)


def _dinv_from(degp_ref):
    deg = jnp.sum(degp_ref[...], axis=0) + 1.0
    dinv = lax.rsqrt(deg)
    rows = lax.iota(jnp.int32, NPAD)
    dinv = jnp.where(rows < N, dinv, 0.0)
    return dinv.reshape(NPAD, 1)


def _tc1_body(x_ref, w_ref, degp_ref, o_ref):
    dinv = _dinv_from(degp_ref)
    h = jnp.dot(x_ref[...], w_ref[...], preferred_element_type=jnp.float32)
    o_ref[...] = h * dinv


def _tc2_body(p_ref, degp_ref, b_ref, w_ref, o_ref):
    dinv = _dinv_from(degp_ref)
    agg = p_ref[0:NPAD, :] + p_ref[NPAD:2 * NPAD, :]
    a = agg * dinv + b_ref[...]
    r = jnp.maximum(a, 0.0)
    h = jnp.dot(r, w_ref[...], preferred_element_type=jnp.float32)
    o_ref[...] = h * dinv


def _tc3_body(p_ref, degp_ref, b_ref, wh_ref, bh_ref, o_ref):
    dinv = _dinv_from(degp_ref)
    agg = p_ref[0:NPAD, :] + p_ref[NPAD:2 * NPAD, :]
    a = agg * dinv + b_ref[...]
    r = jnp.maximum(a, 0.0)
    out = jnp.dot(r, wh_ref[...], preferred_element_type=jnp.float32) + bh_ref[...]
    o_ref[...] = out[0:N, :]


def _tc_call(body, out_shape, *args):
    return pl.pallas_call(body, out_shape=out_shape)(*args)


def kernel(x, edge_index, W1, b1, W2, b2, Wh, bh):
    src = edge_index[0].astype(jnp.int32)
    dst = edge_index[1].astype(jnp.int32)
    padlen = EPAD - E
    padidx = jnp.full((padlen,), N, dtype=jnp.int32)
    src3 = jnp.concatenate([src, padidx]).reshape(TILES * BPT, BLK)
    dst3 = jnp.concatenate([dst, padidx]).reshape(TILES * BPT, BLK)

    x_ext = jnp.concatenate([x, jnp.zeros((NPAD - N, D), jnp.float32)], axis=0)
    zeros_tile = jnp.zeros((BLK, D), jnp.float32)

    degp = _deg(dst3).reshape(TILES, NPAD)

    g1 = _tc_call(_tc1_body, jax.ShapeDtypeStruct((NPAD, D), jnp.float32),
                  x_ext, W1, degp)
    p1 = _agg(g1, src3, dst3, zeros_tile)
    g2 = _tc_call(_tc2_body, jax.ShapeDtypeStruct((NPAD, D), jnp.float32),
                  p1, degp, b1.reshape(1, D), W2)
    p2 = _agg(g2, src3, dst3, zeros_tile)
    out = _tc_call(_tc3_body, jax.ShapeDtypeStruct((N, bh.shape[0]), jnp.float32),
                   p2, degp, b2.reshape(1, D), Wh, bh.reshape(1, -1))
    return out


# speedup vs baseline: 2.4471x; 2.4471x over previous
import jax
import jax.numpy as jnp
from jax import lax
from jax.experimental import pallas as pl
from jax.experimental.pallas import tpu as pltpu
from jax.experimental.pallas import tpu_sc as plsc

N = 10000
NPAD = 10240
E = 320000
D = 128
NC, NS = 2, 16
TILES = NC * NS
BLK = 80
BPT = 128
EPAD = TILES * BPT * BLK
ROWS_PER_TILE = NPAD // NS


def _sc_mesh():
    return plsc.VectorSubcoreMesh(core_axis_name="c", subcore_axis_name="s")


def _deg_kernel_body(dst_hbm, out_hbm, hist_v, idx_v):
    c = lax.axis_index("c")
    s = lax.axis_index("s")
    wid = c * NS + s

    pltpu.sync_copy(dst_hbm.at[pl.ds(wid * BPT, BPT)], idx_v)

    @pl.loop(0, NPAD // 16)
    def _(i):
        hist_v[pl.ds(i * 16, 16)] = jnp.zeros((16,), jnp.float32)

    @pl.loop(0, BPT)
    def _(b):
        for j in range(BLK // 16):
            idx = idx_v[b, pl.ds(j * 16, 16)]
            cnt, last = plsc.scan_count(idx)
            plsc.addupdate_scatter(hist_v, [idx], cnt.astype(jnp.float32),
                                   mask=last)

    pltpu.sync_copy(hist_v, out_hbm.at[pl.ds(wid * NPAD, NPAD)])


def _deg(dst3):
    fn = pl.kernel(
        _deg_kernel_body,
        out_type=jax.ShapeDtypeStruct((TILES * NPAD,), jnp.float32),
        mesh=_sc_mesh(),
        compiler_params=pltpu.CompilerParams(needs_layout_passes=False),
        scratch_types=[
            pltpu.VMEM((NPAD,), jnp.float32),
            pltpu.VMEM((BPT, BLK), jnp.int32),
        ],
    )
    return fn(dst3)


def _agg_kernel_body(g_hbm, src_hbm, dst_hbm, zeros_hbm, out_hbm,
                     acc, sidx, didx, rows0, sem0, sems):
    c = lax.axis_index("c")
    s = lax.axis_index("s")
    wid = c * NS + s
    row0 = s * ROWS_PER_TILE

    pltpu.sync_copy(src_hbm.at[pl.ds(wid * BPT, BPT)], sidx)
    pltpu.sync_copy(dst_hbm.at[pl.ds(wid * BPT, BPT)], didx)

    @pl.when(c == 0)
    def _():
        @pl.loop(0, ROWS_PER_TILE // BLK)
        def _(k):
            pltpu.sync_copy(g_hbm.at[pl.ds(row0 + k * BLK, BLK)], rows0.at[pl.ds(0, BLK)])
            pltpu.sync_copy(rows0.at[pl.ds(0, BLK)], acc.at[pl.ds(row0 + k * BLK, BLK)])

    @pl.when(c == 1)
    def _():
        pltpu.sync_copy(zeros_hbm, rows0.at[pl.ds(0, BLK)])

        @pl.loop(0, ROWS_PER_TILE // BLK)
        def _(k):
            pltpu.sync_copy(rows0.at[pl.ds(0, BLK)], acc.at[pl.ds(row0 + k * BLK, BLK)])

    plsc.subcore_barrier()

    @pl.loop(0, BPT)
    def _(b):
        cur = lax.rem(b, 2) * BLK
        prv = (1 - lax.rem(b, 2)) * BLK
        pltpu.async_copy(g_hbm.at[sidx.at[b]], rows0.at[pl.ds(cur, BLK)],
                         sem0).wait()

        del prv

        @pl.when(b > 0)
        def _():
            pltpu.make_async_copy(zeros_hbm, acc.at[pl.ds(0, BLK)], sems).wait()

        pltpu.async_copy(rows0.at[pl.ds(cur, BLK)], acc.at[didx.at[b]],
                         sems, add=True)

    pltpu.make_async_copy(zeros_hbm, acc.at[pl.ds(0, BLK)], sems).wait()

    plsc.subcore_barrier()

    @pl.loop(0, ROWS_PER_TILE // BLK)
    def _(k):
        pltpu.sync_copy(acc.at[pl.ds(row0 + k * BLK, BLK)], rows0.at[pl.ds(0, BLK)])
        pltpu.sync_copy(rows0.at[pl.ds(0, BLK)], out_hbm.at[pl.ds(c * NPAD + row0 + k * BLK, BLK)])


def _agg(g, src3, dst3, zeros_tile):
    fn = pl.kernel(
        _agg_kernel_body,
        out_type=jax.ShapeDtypeStruct((NC * NPAD, D), jnp.float32),
        mesh=_sc_mesh(),
        compiler_params=pltpu.CompilerParams(use_tc_tiling_on_sc=False),
        scratch_types=[
            pltpu.VMEM_SHARED((NPAD, D), jnp.float32),
            pltpu.VMEM((BPT, BLK), jnp.int32),
            pltpu.VMEM((BPT, BLK), jnp.int32),
            pltpu.VMEM((2 * BLK, D), jnp.float32),
            pltpu.SemaphoreType.DMA,
            pltpu.SemaphoreType.DMA,
        ],
    )
    return fn(g, src3, dst3, zeros_tile)


def _dinv_from(degp_ref):
    deg = jnp.sum(degp_ref[...], axis=0) + 1.0
    dinv = lax.rsqrt(deg)
    rows = lax.iota(jnp.int32, NPAD)
    dinv = jnp.where(rows < N, dinv, 0.0)
    return dinv.reshape(NPAD, 1)


def _tc1_body(x_ref, w_ref, degp_ref, o_ref):
    dinv = _dinv_from(degp_ref)
    h = jnp.dot(x_ref[...], w_ref[...], preferred_element_type=jnp.float32)
    o_ref[...] = h * dinv


def _tc2_body(p_ref, degp_ref, b_ref, w_ref, o_ref):
    dinv = _dinv_from(degp_ref)
    agg = p_ref[0:NPAD, :] + p_ref[NPAD:2 * NPAD, :]
    a = agg * dinv + b_ref[...]
    r = jnp.maximum(a, 0.0)
    h = jnp.dot(r, w_ref[...], preferred_element_type=jnp.float32)
    o_ref[...] = h * dinv


def _tc3_body(p_ref, degp_ref, b_ref, wh_ref, bh_ref, o_ref):
    dinv = _dinv_from(degp_ref)
    agg = p_ref[0:NPAD, :] + p_ref[NPAD:2 * NPAD, :]
    a = agg * dinv + b_ref[...]
    r = jnp.maximum(a, 0.0)
    out = jnp.dot(r, wh_ref[...], preferred_element_type=jnp.float32) + bh_ref[...]
    o_ref[...] = out[0:N, :]


def _tc_call(body, out_shape, *args):
    return pl.pallas_call(body, out_shape=out_shape)(*args)


def kernel(x, edge_index, W1, b1, W2, b2, Wh, bh):
    src = edge_index[0].astype(jnp.int32)
    dst = edge_index[1].astype(jnp.int32)
    padlen = EPAD - E
    padidx = N + (jnp.arange(padlen, dtype=jnp.int32) % (NPAD - N))
    src3 = jnp.concatenate([src, padidx]).reshape(TILES * BPT, BLK)
    dst3 = jnp.concatenate([dst, padidx]).reshape(TILES * BPT, BLK)

    x_ext = jnp.concatenate([x, jnp.zeros((NPAD - N, D), jnp.float32)], axis=0)
    zeros_tile = jnp.zeros((BLK, D), jnp.float32)

    degp = _deg(dst3).reshape(TILES, NPAD)

    g1 = _tc_call(_tc1_body, jax.ShapeDtypeStruct((NPAD, D), jnp.float32),
                  x_ext, W1, degp)
    p1 = _agg(g1, src3, dst3, zeros_tile)
    g2 = _tc_call(_tc2_body, jax.ShapeDtypeStruct((NPAD, D), jnp.float32),
                  p1, degp, b1.reshape(1, D), W2)
    p2 = _agg(g2, src3, dst3, zeros_tile)
    out = _tc_call(_tc3_body, jax.ShapeDtypeStruct((N, bh.shape[0]), jnp.float32),
                   p2, degp, b2.reshape(1, D), Wh, bh.reshape(1, -1))
    return out
